# baseline (speedup 1.0000x reference)
import functools

import jax
import jax.numpy as jnp
from jax import lax
from jax.experimental import pallas as pl
from jax.experimental.pallas import tpu as pltpu
from jax.experimental.pallas import tpu_sc as plsc

N = 100000
E = 1600000
L = 16
NC = 2
NS = 16
G = 8
CR = 5
CE = CR * 128
ROWS_PAD = 12800
E_PAD = ROWS_PAD * 128
ACC_ROWS = 100096
ZROWS = ACC_ROWS // NS
F = 16

_sc_mesh = plsc.VectorSubcoreMesh(core_axis_name="c", subcore_axis_name="s")


def _deg_body(srcp, dstp, ones, zeros, degp0, degp1, dloc2d,
              src_v, dst_v, sloc_v, dloc_v, ones_v, acc):
    c = lax.axis_index("c")
    s = lax.axis_index("s")
    wid = c * NS + s

    pltpu.sync_copy(zeros, acc.at[pl.ds(s * ZROWS, ZROWS)])
    pltpu.sync_copy(ones, ones_v)
    plsc.subcore_barrier()

    trash = jnp.full((L,), N, jnp.int32) + s

    rows_per_tile = ROWS_PAD // (NC * NS)
    row_base = wid * rows_per_tile

    def chunk(it, carry):
        r0 = row_base + it * G
        pltpu.sync_copy(srcp.at[pl.ds(r0, G)], src_v)
        pltpu.sync_copy(dstp.at[pl.ds(r0, G)], dst_v)
        for j in range(G):
            for i in range(128 // L):
                sl = pl.ds(i * L, L)
                s16 = src_v[j, sl]
                d16 = dst_v[j, sl]
                eq = s16 == d16
                sloc_v[j, sl] = jnp.where(eq, trash, s16)
                dloc_v[j, sl] = jnp.where(eq, trash, d16)
        pltpu.sync_copy(dloc_v, dloc2d.at[pl.ds(r0, G)])
        for j in range(G):
            pltpu.sync_copy(ones_v, acc.at[sloc_v.at[j]], add=True)
        return carry

    lax.fori_loop(0, rows_per_tile // G, chunk, 0)

    plsc.subcore_barrier()

    @pl.when(c == 0)
    def _():
        pltpu.sync_copy(acc.at[pl.ds(s * ZROWS, ZROWS)],
                        degp0.at[pl.ds(s * ZROWS, ZROWS)])

    @pl.when(c == 1)
    def _():
        pltpu.sync_copy(acc.at[pl.ds(s * ZROWS, ZROWS)],
                        degp1.at[pl.ds(s * ZROWS, ZROWS)])


_deg_call = pl.kernel(
    _deg_body,
    out_type=(
        jax.ShapeDtypeStruct((ACC_ROWS, F), jnp.float32),
        jax.ShapeDtypeStruct((ACC_ROWS, F), jnp.float32),
        jax.ShapeDtypeStruct((ROWS_PAD, 128), jnp.int32),
    ),
    mesh=_sc_mesh,
    compiler_params=pltpu.CompilerParams(use_tc_tiling_on_sc=False),
    scratch_types=[
        pltpu.VMEM((G, 128), jnp.int32),
        pltpu.VMEM((G, 128), jnp.int32),
        pltpu.VMEM((G, 128), jnp.int32),
        pltpu.VMEM((G, 128), jnp.int32),
        pltpu.VMEM((128, F), jnp.float32),
        pltpu.VMEM_SHARED((ACC_ROWS, F), jnp.float32),
    ],
)


def _prop_body(srcp, dlocp, glo, ghi, zeros, plo, phi,
               src_v, dloc_v, rows_v, acc, sem_i, sem_g, sem_s):
    c = lax.axis_index("c")
    s = lax.axis_index("s")

    pltpu.sync_copy(zeros, acc.at[pl.ds(s * ZROWS, ZROWS)])
    plsc.subcore_barrier()

    rows_per_tile = ROWS_PAD // NS
    row_base = s * rows_per_tile
    n_chunks = rows_per_tile // CR
    nb = n_chunks // 2

    def make_loop(g_ref, sem_i):
        def e0_of(ck):
            return (row_base + ck * CR) * 128

        def fire_idx(ck, p):
            return (pltpu.async_copy(srcp.at[pl.ds(e0_of(ck), CE)],
                                     src_v.at[p], sem_i),
                    pltpu.async_copy(dlocp.at[pl.ds(e0_of(ck), CE)],
                                    dloc_v.at[p], sem_i))

        def wait_idx_recon(ck, p):
            pltpu.make_async_copy(srcp.at[pl.ds(e0_of(ck), CE)],
                                  src_v.at[p], sem_i).wait()
            pltpu.make_async_copy(dlocp.at[pl.ds(e0_of(ck), CE)],
                                  dloc_v.at[p], sem_i).wait()

        def fire_gathers(p):
            return [pltpu.async_copy(g_ref.at[src_v.at[p]],
                                     rows_v.at[p], sem_g)]

        def fire_scatters(p):
            return [pltpu.async_copy(rows_v.at[p],
                                     acc.at[dloc_v.at[p]], sem_s,
                                     add=True)]

        def wait_scatters_recon(p):
            pltpu.make_async_copy(rows_v.at[p],
                                  acc.at[dloc_v.at[p]],
                                  sem_s).wait()

        fire_idx(0, 0)

        def body(q, carry):
            c0 = 2 * q
            wait_idx_recon(c0, 0)
            di1 = fire_idx(c0 + 1, 1)
            gd0 = fire_gathers(0)

            @pl.when(q > 0)
            def _():
                wait_scatters_recon(1)
            for d in gd0:
                d.wait()
            sd0 = fire_scatters(0)
            for d in di1:
                d.wait()

            @pl.when(q < nb - 1)
            def _():
                fire_idx(c0 + 2, 0)
            gd1 = fire_gathers(1)
            for d in sd0:
                d.wait()
            for d in gd1:
                d.wait()
            fire_scatters(1)
            return carry

        lax.fori_loop(0, nb, body, 0)
        wait_scatters_recon(1)

    @pl.when(c == 0)
    def _():
        make_loop(glo, sem_i)

    @pl.when(c == 1)
    def _():
        make_loop(ghi, sem_i)

    plsc.subcore_barrier()

    @pl.when(c == 0)
    def _():
        pltpu.sync_copy(acc.at[pl.ds(s * ZROWS, ZROWS)],
                        plo.at[pl.ds(s * ZROWS, ZROWS)])

    @pl.when(c == 1)
    def _():
        pltpu.sync_copy(acc.at[pl.ds(s * ZROWS, ZROWS)],
                        phi.at[pl.ds(s * ZROWS, ZROWS)])


_prop_call = pl.kernel(
    _prop_body,
    out_type=(
        jax.ShapeDtypeStruct((ACC_ROWS, F), jnp.float32),
        jax.ShapeDtypeStruct((ACC_ROWS, F), jnp.float32),
    ),
    mesh=_sc_mesh,
    compiler_params=pltpu.CompilerParams(use_tc_tiling_on_sc=False),
    scratch_types=[
        pltpu.VMEM((2, CE), jnp.int32),
        pltpu.VMEM((2, CE), jnp.int32),
        pltpu.VMEM((2, CE, F), jnp.float32),
        pltpu.VMEM_SHARED((ACC_ROWS, F), jnp.float32),
        pltpu.SemaphoreType.DMA,
        pltpu.SemaphoreType.DMA,
        pltpu.SemaphoreType.DMA,
    ],
)


_B = 2000


def _tc0_body(degp0, degp1, xp, W, b, h_out, glo_out, ghi_out, dinv8_out):
    deg = degp0[:, 0:1] + degp1[:, 0:1]
    dinv = jnp.where(deg > 0, lax.rsqrt(jnp.where(deg > 0, deg, 1.0)), 0.0)
    h = b[...]
    h = h + xp[:, 0:1] * W[0:1, :]
    h = h + xp[:, 1:2] * W[1:2, :]
    h = h + xp[:, 2:3] * W[2:3, :]
    h = jnp.maximum(h, 0.0)
    g = dinv * h
    h_out[...] = h
    glo_out[...] = g[:, :F]
    ghi_out[...] = g[:, F:]
    dinv8_out[...] = jnp.broadcast_to(dinv, (_B, 8))


def _conv_math(h, plo, phi, dinv8, W0, W1, b, res):
    dinv = dinv8[:, 0:1]
    tlo = (-dinv) * plo[...]
    thi = (-dinv) * phi[...]
    acc = jnp.dot(h[...], W0[...], preferred_element_type=jnp.float32)
    acc = acc + jnp.dot(tlo, W1[0:F, :], preferred_element_type=jnp.float32)
    acc = acc + jnp.dot(thi, W1[F:, :], preferred_element_type=jnp.float32)
    acc = acc + b[...]
    if res is not None:
        acc = acc + res[...]
    return acc


def _conv_body(h, plo, phi, dinv8, W0, W1, b, h_out, glo_out, ghi_out):
    hn = jnp.maximum(_conv_math(h, plo, phi, dinv8, W0, W1, b, None), 0.0)
    g = dinv8[:, 0:1] * hn
    h_out[...] = hn
    glo_out[...] = g[:, :F]
    ghi_out[...] = g[:, F:]


def _conv_res_body(h, plo, phi, dinv8, W0, W1, b, res, h_out, glo_out, ghi_out):
    hn = jnp.maximum(_conv_math(h, plo, phi, dinv8, W0, W1, b, res), 0.0)
    g = dinv8[:, 0:1] * hn
    h_out[...] = hn
    glo_out[...] = g[:, :F]
    ghi_out[...] = g[:, F:]


def _conv_final_body(h, plo, phi, dinv8, W0, W1, b, res, lw, lb, out8):
    hn = jnp.maximum(_conv_math(h, plo, phi, dinv8, W0, W1, b, res), 0.0)
    out8[...] = jnp.dot(hn, lw[...], preferred_element_type=jnp.float32) + lb[...]


def _rows(shape):
    return pl.BlockSpec(shape, lambda i: (i, 0))


def _whole(shape):
    return pl.BlockSpec(shape, lambda i: (0, 0))


_grid = N // _B


def _tc0(degp0, degp1, xp, W, b):
    return pl.pallas_call(
        _tc0_body,
        grid=(_grid,),
        in_specs=[_rows((_B, F)), _rows((_B, F)), _rows((_B, 8)),
                  _whole((8, 32)), _whole((1, 32))],
        out_specs=[_rows((_B, 32)), _rows((_B, F)), _rows((_B, F)),
                   _rows((_B, 8))],
        out_shape=[
            jax.ShapeDtypeStruct((N, 32), jnp.float32),
            jax.ShapeDtypeStruct((N, F), jnp.float32),
            jax.ShapeDtypeStruct((N, F), jnp.float32),
            jax.ShapeDtypeStruct((N, 8), jnp.float32),
        ],
    )(degp0, degp1, xp, W, b)


def _conv(h, plo, phi, dinv8, W0, W1, b):
    return pl.pallas_call(
        _conv_body,
        grid=(_grid,),
        in_specs=[_rows((_B, 32)), _rows((_B, F)), _rows((_B, F)),
                  _rows((_B, 8)), _whole((32, 32)), _whole((32, 32)),
                  _whole((1, 32))],
        out_specs=[_rows((_B, 32)), _rows((_B, F)), _rows((_B, F))],
        out_shape=[
            jax.ShapeDtypeStruct((N, 32), jnp.float32),
            jax.ShapeDtypeStruct((N, F), jnp.float32),
            jax.ShapeDtypeStruct((N, F), jnp.float32),
        ],
    )(h, plo, phi, dinv8, W0, W1, b)


def _conv_res(h, plo, phi, dinv8, W0, W1, b, res):
    return pl.pallas_call(
        _conv_res_body,
        grid=(_grid,),
        in_specs=[_rows((_B, 32)), _rows((_B, F)), _rows((_B, F)),
                  _rows((_B, 8)), _whole((32, 32)), _whole((32, 32)),
                  _whole((1, 32)), _rows((_B, 32))],
        out_specs=[_rows((_B, 32)), _rows((_B, F)), _rows((_B, F))],
        out_shape=[
            jax.ShapeDtypeStruct((N, 32), jnp.float32),
            jax.ShapeDtypeStruct((N, F), jnp.float32),
            jax.ShapeDtypeStruct((N, F), jnp.float32),
        ],
    )(h, plo, phi, dinv8, W0, W1, b, res)


def _conv_final(h, plo, phi, dinv8, W0, W1, b, res, lw, lb):
    return pl.pallas_call(
        _conv_final_body,
        grid=(_grid,),
        in_specs=[_rows((_B, 32)), _rows((_B, F)), _rows((_B, F)),
                  _rows((_B, 8)), _whole((32, 32)), _whole((32, 32)),
                  _whole((1, 32)), _rows((_B, 32)), _whole((32, 8)),
                  _whole((1, 8))],
        out_specs=[_rows((_B, 8))],
        out_shape=[jax.ShapeDtypeStruct((N, 8), jnp.float32)],
    )(h, plo, phi, dinv8, W0, W1, b, res, lw, lb)


@jax.jit
def _run(x, edge_index, lin0_W, lin0_b, c11_W0, c11_W1, c11_b,
         c12_W0, c12_W1, c12_b, c21_W0, c21_W1, c21_b,
         c22_W0, c22_W1, c22_b, lin1_W, lin1_b):
    pad = E_PAD - E
    srcp = jnp.pad(edge_index[0], (0, pad)).reshape(ROWS_PAD, 128)
    dstp = jnp.pad(edge_index[1], (0, pad)).reshape(ROWS_PAD, 128)
    ones = jnp.ones((128, F), jnp.float32)
    zeros = jnp.zeros((ZROWS, F), jnp.float32)

    degp0, degp1, dloc2d = _deg_call(srcp, dstp, ones, zeros)
    src_flat = srcp.reshape(-1)
    dloc_flat = dloc2d.reshape(-1)

    xp = jnp.pad(x, ((0, 0), (0, 5)))
    W0p = jnp.pad(lin0_W, ((0, 5), (0, 0)))
    h1, g_lo, g_hi, dinv8 = _tc0(degp0[:N], degp1[:N], xp, W0p,
                                 lin0_b.reshape(1, 32))

    plo, phi = _prop_call(src_flat, dloc_flat, g_lo, g_hi, zeros)
    plo, phi = plo[:N], phi[:N]
    h2, g_lo, g_hi = _conv(h1, plo, phi, dinv8, c11_W0, c11_W1,
                           c11_b.reshape(1, 32))

    plo, phi = _prop_call(src_flat, dloc_flat, g_lo, g_hi, zeros)
    plo, phi = plo[:N], phi[:N]
    h3, g_lo, g_hi = _conv_res(h2, plo, phi, dinv8, c12_W0, c12_W1,
                               c12_b.reshape(1, 32), h1)

    plo, phi = _prop_call(src_flat, dloc_flat, g_lo, g_hi, zeros)
    plo, phi = plo[:N], phi[:N]
    h4, g_lo, g_hi = _conv(h3, plo, phi, dinv8, c21_W0, c21_W1,
                           c21_b.reshape(1, 32))

    plo, phi = _prop_call(src_flat, dloc_flat, g_lo, g_hi, zeros)
    plo, phi = plo[:N], phi[:N]
    lwp = jnp.pad(lin1_W, ((0, 0), (0, 7)))
    lbp = jnp.pad(lin1_b, (0, 7)).reshape(1, 8)
    out8, = _conv_final(h4, plo, phi, dinv8, c22_W0, c22_W1,
                        c22_b.reshape(1, 32), h3, lwp, lbp)
    return out8[:, :1]


def kernel(x, edge_index, lin0_W, lin0_b, c11_W0, c11_W1, c11_b,
           c12_W0, c12_W1, c12_b, c21_W0, c21_W1, c21_b,
           c22_W0, c22_W1, c22_b, lin1_W, lin1_b):
    return _run(x, edge_index, lin0_W, lin0_b, c11_W0, c11_W1, c11_b,
                c12_W0, c12_W1, c12_b, c21_W0, c21_W1, c21_b,
                c22_W0, c22_W1, c22_b, lin1_W, lin1_b)


# speedup vs baseline: 1.3185x; 1.3185x over previous
import jax
import jax.numpy as jnp
from jax import lax
from jax.experimental import pallas as pl
from jax.experimental.pallas import tpu as pltpu
from jax.experimental.pallas import tpu_sc as plsc

N = 100000
E = 1600000
L = 16
NC = 2
NS = 16
E_PAD = 1638400
ACC_ROWS = 100096
ZROWS = ACC_ROWS // NS
F = 16
CE = 640
RPK = ACC_ROWS // 8

_sc_mesh = plsc.VectorSubcoreMesh(core_axis_name="c", subcore_axis_name="s")


def _deg_body(srcp, dstp, ones, zeros, degp0, degp1, dloc_o, slo_o, shi_o,
              src_v, dst_v, sloc_v, dloc_v, slo_v, shi_v, ones_v, acc):
    c = lax.axis_index("c")
    s = lax.axis_index("s")
    wid = c * NS + s

    pltpu.sync_copy(zeros, acc.at[pl.ds(s * ZROWS, ZROWS)])
    pltpu.sync_copy(ones, ones_v)
    plsc.subcore_barrier()

    trash = jnp.full((L,), N, jnp.int32) + s
    trash2 = jnp.full((L,), 0, jnp.int32)

    edges_per_tile = E_PAD // (NC * NS)
    e_base = wid * edges_per_tile

    def chunk(it, carry):
        e0 = e_base + it * CE
        pltpu.sync_copy(srcp.at[pl.ds(e0, CE)], src_v)
        pltpu.sync_copy(dstp.at[pl.ds(e0, CE)], dst_v)
        for i in range(CE // L):
            sl = pl.ds(i * L, L)
            s16 = src_v[sl]
            d16 = dst_v[sl]
            eq = s16 == d16
            sloc_v[sl] = jnp.where(eq, trash, s16)
            dloc_v[sl] = jnp.where(eq, trash, d16)
            s2 = s16 + s16
            slo_v[sl] = jnp.where(eq, trash2, s2)
            shi_v[sl] = jnp.where(eq, trash2, s2 + 1)
        pltpu.sync_copy(dloc_v, dloc_o.at[pl.ds(e0, CE)])
        pltpu.sync_copy(slo_v, slo_o.at[pl.ds(e0, CE)])
        pltpu.sync_copy(shi_v, shi_o.at[pl.ds(e0, CE)])
        pltpu.sync_copy(ones_v, acc.at[sloc_v], add=True)
        return carry

    lax.fori_loop(0, edges_per_tile // CE, chunk, 0)

    plsc.subcore_barrier()

    @pl.when(c == 0)
    def _():
        pltpu.sync_copy(acc.at[pl.ds(s * ZROWS, ZROWS)],
                        degp0.at[pl.ds(s * ZROWS, ZROWS)])

    @pl.when(c == 1)
    def _():
        pltpu.sync_copy(acc.at[pl.ds(s * ZROWS, ZROWS)],
                        degp1.at[pl.ds(s * ZROWS, ZROWS)])


_deg_call = pl.kernel(
    _deg_body,
    out_type=(
        jax.ShapeDtypeStruct((ACC_ROWS, F), jnp.float32),
        jax.ShapeDtypeStruct((ACC_ROWS, F), jnp.float32),
        jax.ShapeDtypeStruct((E_PAD,), jnp.int32),
        jax.ShapeDtypeStruct((E_PAD,), jnp.int32),
        jax.ShapeDtypeStruct((E_PAD,), jnp.int32),
    ),
    mesh=_sc_mesh,
    compiler_params=pltpu.CompilerParams(use_tc_tiling_on_sc=False),
    scratch_types=[
        pltpu.VMEM((CE,), jnp.int32),
        pltpu.VMEM((CE,), jnp.int32),
        pltpu.VMEM((CE,), jnp.int32),
        pltpu.VMEM((CE,), jnp.int32),
        pltpu.VMEM((CE,), jnp.int32),
        pltpu.VMEM((CE,), jnp.int32),
        pltpu.VMEM((CE, F), jnp.float32),
        pltpu.VMEM_SHARED((ACC_ROWS, F), jnp.float32),
    ],
)


def _prop_body(g2n, slo, shi, dlocp, zeros, plo, phi,
               src_v, dloc_v, rows_v, acc, sem_i, sem_g, sem_s):
    c = lax.axis_index("c")
    s = lax.axis_index("s")

    pltpu.sync_copy(zeros, acc.at[pl.ds(s * ZROWS, ZROWS)])
    plsc.subcore_barrier()

    edges_per_tile = E_PAD // NS
    e_base = s * edges_per_tile
    n_chunks = edges_per_tile // CE
    nb = n_chunks // 2

    def make_loop(idxp):
        def e0_of(ck):
            return e_base + ck * CE

        def fire_idx(ck, p):
            return (pltpu.async_copy(idxp.at[pl.ds(e0_of(ck), CE)],
                                     src_v.at[p], sem_i),
                    pltpu.async_copy(dlocp.at[pl.ds(e0_of(ck), CE)],
                                    dloc_v.at[p], sem_i))

        def wait_idx_recon(ck, p):
            pltpu.make_async_copy(idxp.at[pl.ds(e0_of(ck), CE)],
                                  src_v.at[p], sem_i).wait()
            pltpu.make_async_copy(dlocp.at[pl.ds(e0_of(ck), CE)],
                                  dloc_v.at[p], sem_i).wait()

        def fire_gather(p):
            return pltpu.async_copy(g2n.at[src_v.at[p]], rows_v.at[p], sem_g)

        def fire_scatter(p):
            return pltpu.async_copy(rows_v.at[p], acc.at[dloc_v.at[p]],
                                    sem_s, add=True)

        def wait_scatter_recon(p):
            pltpu.make_async_copy(rows_v.at[p], acc.at[dloc_v.at[p]],
                                  sem_s).wait()

        fire_idx(0, 0)

        def body(q, carry):
            c0 = 2 * q
            wait_idx_recon(c0, 0)
            di1 = fire_idx(c0 + 1, 1)
            gd0 = fire_gather(0)

            @pl.when(q > 0)
            def _():
                wait_scatter_recon(1)
            gd0.wait()
            sd0 = fire_scatter(0)
            for d in di1:
                d.wait()

            @pl.when(q < nb - 1)
            def _():
                fire_idx(c0 + 2, 0)
            gd1 = fire_gather(1)
            sd0.wait()
            gd1.wait()
            fire_scatter(1)
            return carry

        lax.fori_loop(0, nb, body, 0)
        wait_scatter_recon(1)

    @pl.when(c == 0)
    def _():
        make_loop(slo)

    @pl.when(c == 1)
    def _():
        make_loop(shi)

    plsc.subcore_barrier()

    @pl.when(c == 0)
    def _():
        pltpu.sync_copy(acc.at[pl.ds(s * ZROWS, ZROWS)],
                        plo.at[pl.ds(s * ZROWS, ZROWS)])

    @pl.when(c == 1)
    def _():
        pltpu.sync_copy(acc.at[pl.ds(s * ZROWS, ZROWS)],
                        phi.at[pl.ds(s * ZROWS, ZROWS)])


_prop_call = pl.kernel(
    _prop_body,
    out_type=(
        jax.ShapeDtypeStruct((ACC_ROWS, F), jnp.float32),
        jax.ShapeDtypeStruct((ACC_ROWS, F), jnp.float32),
    ),
    mesh=_sc_mesh,
    compiler_params=pltpu.CompilerParams(use_tc_tiling_on_sc=False),
    scratch_types=[
        pltpu.VMEM((2, CE), jnp.int32),
        pltpu.VMEM((2, CE), jnp.int32),
        pltpu.VMEM((2, CE, F), jnp.float32),
        pltpu.VMEM_SHARED((ACC_ROWS, F), jnp.float32),
        pltpu.SemaphoreType.DMA,
        pltpu.SemaphoreType.DMA,
        pltpu.SemaphoreType.DMA,
    ],
)


_BB = 3128


def _rowspec(width):
    return pl.BlockSpec((_BB, width), lambda i: (i, 0))


def _whole(shape):
    return pl.BlockSpec(shape, lambda i: (0, 0))


def _dinv_body(degp0, degp1, dinv128_out):
    deg = degp0[...] + degp1[...]
    dinv128_out[...] = jnp.where(deg > 0,
                                 lax.rsqrt(jnp.where(deg > 0, deg, 1.0)), 0.0)


def _tc_dinv(degp0_pk, degp1_pk):
    return pl.pallas_call(
        _dinv_body,
        grid=(4,),
        in_specs=[pl.BlockSpec((_BB, 128), lambda i: (i, 0))] * 2,
        out_specs=pl.BlockSpec((_BB, 128), lambda i: (i, 0)),
        out_shape=jax.ShapeDtypeStruct((RPK, 128), jnp.float32),
    )(degp0_pk, degp1_pk)


def _tc0_body(xpk, dinv256, Wx, b, h_out, g_out):
    h = jnp.dot(xpk[...], Wx[...], preferred_element_type=jnp.float32) + b[...]
    h = jnp.maximum(h, 0.0)
    h_out[...] = h
    g_out[...] = dinv256[...] * h


def _tc0(xpk, dinv256, Wx, b):
    return pl.pallas_call(
        _tc0_body,
        grid=(RPK // _BB,),
        in_specs=[_rowspec(64), _rowspec(256), _whole((64, 256)),
                  _whole((1, 256))],
        out_specs=[_rowspec(256), _rowspec(256)],
        out_shape=[
            jax.ShapeDtypeStruct((RPK, 256), jnp.float32),
            jax.ShapeDtypeStruct((RPK, 256), jnp.float32),
        ],
    )(xpk, dinv256, Wx, b)


def _conv_math(h, plo, phi, dinv128, W0, W1a, W1b, b, res):
    tlo = (-dinv128[...]) * plo[...]
    thi = (-dinv128[...]) * phi[...]
    acc = jnp.dot(h[...], W0[...], preferred_element_type=jnp.float32)
    acc = acc + jnp.dot(tlo, W1a[...], preferred_element_type=jnp.float32)
    acc = acc + jnp.dot(thi, W1b[...], preferred_element_type=jnp.float32)
    acc = acc + b[...]
    if res is not None:
        acc = acc + res[...]
    return jnp.maximum(acc, 0.0)


def _conv_body(h, plo, phi, dinv128, dinv256, W0, W1a, W1b, b, h_out, g_out):
    hn = _conv_math(h, plo, phi, dinv128, W0, W1a, W1b, b, None)
    h_out[...] = hn
    g_out[...] = dinv256[...] * hn


def _conv_res_body(h, plo, phi, dinv128, dinv256, W0, W1a, W1b, b, res,
                   h_out, g_out):
    hn = _conv_math(h, plo, phi, dinv128, W0, W1a, W1b, b, res)
    h_out[...] = hn
    g_out[...] = dinv256[...] * hn


def _conv_final_body(h, plo, phi, dinv128, W0, W1a, W1b, b, res, lw, lb,
                     out_pk):
    hn = _conv_math(h, plo, phi, dinv128, W0, W1a, W1b, b, res)
    out_pk[...] = jnp.dot(hn, lw[...],
                          preferred_element_type=jnp.float32) + lb[...]


_conv_common_in = [_rowspec(256), _rowspec(128), _rowspec(128),
                   _rowspec(128), _rowspec(256), _whole((256, 256)),
                   _whole((128, 256)), _whole((128, 256)), _whole((1, 256))]


def _conv(h, plo, phi, dinv128, dinv256, W0, W1a, W1b, b):
    return pl.pallas_call(
        _conv_body,
        grid=(RPK // _BB,),
        in_specs=_conv_common_in,
        out_specs=[_rowspec(256), _rowspec(256)],
        out_shape=[
            jax.ShapeDtypeStruct((RPK, 256), jnp.float32),
            jax.ShapeDtypeStruct((RPK, 256), jnp.float32),
        ],
    )(h, plo, phi, dinv128, dinv256, W0, W1a, W1b, b)


def _conv_res(h, plo, phi, dinv128, dinv256, W0, W1a, W1b, b, res):
    return pl.pallas_call(
        _conv_res_body,
        grid=(RPK // _BB,),
        in_specs=_conv_common_in + [_rowspec(256)],
        out_specs=[_rowspec(256), _rowspec(256)],
        out_shape=[
            jax.ShapeDtypeStruct((RPK, 256), jnp.float32),
            jax.ShapeDtypeStruct((RPK, 256), jnp.float32),
        ],
    )(h, plo, phi, dinv128, dinv256, W0, W1a, W1b, b, res)


def _conv_final(h, plo, phi, dinv128, W0, W1a, W1b, b, res, lw, lb):
    return pl.pallas_call(
        _conv_final_body,
        grid=(RPK // _BB,),
        in_specs=[_rowspec(256), _rowspec(128), _rowspec(128),
                  _rowspec(128), _whole((256, 256)), _whole((128, 256)),
                  _whole((128, 256)), _whole((1, 256)), _rowspec(256),
                  _whole((256, 8)), _whole((1, 8))],
        out_specs=[_rowspec(8)],
        out_shape=[jax.ShapeDtypeStruct((RPK, 8), jnp.float32)],
    )(h, plo, phi, dinv128, W0, W1a, W1b, b, res, lw, lb)


def _kron8(W):
    k, m = W.shape
    eye = jnp.eye(8, dtype=jnp.float32)
    return (eye[:, None, :, None] * W[None, :, None, :]).reshape(8 * k, 8 * m)


@jax.jit
def _run(x, edge_index, lin0_W, lin0_b, c11_W0, c11_W1, c11_b,
         c12_W0, c12_W1, c12_b, c21_W0, c21_W1, c21_b,
         c22_W0, c22_W1, c22_b, lin1_W, lin1_b):
    pad = E_PAD - E
    src1 = jnp.pad(edge_index[0], (0, pad))
    dst1 = jnp.pad(edge_index[1], (0, pad))
    ones = jnp.ones((CE, F), jnp.float32)
    zeros = jnp.zeros((ZROWS, F), jnp.float32)

    degp0, degp1, dloc, slo, shi = _deg_call(src1, dst1, ones, zeros)

    dinv128 = _tc_dinv(degp0.reshape(RPK, 128), degp1.reshape(RPK, 128))
    dinv_n = dinv128.reshape(ACC_ROWS, F)[:, :1]
    dinv256 = jnp.broadcast_to(dinv_n, (ACC_ROWS, 32)).reshape(RPK, 256)

    xpk = jnp.pad(x, ((0, ACC_ROWS - N), (0, 5))).reshape(RPK, 64)
    Wx = _kron8(jnp.pad(lin0_W, ((0, 5), (0, 0))))
    bpk = jnp.tile(lin0_b, 8).reshape(1, 256)
    h1, g = _tc0(xpk, dinv256, Wx, bpk)

    def wpk(W0, W1, b):
        return (_kron8(W0), _kron8(W1[:F, :]), _kron8(W1[F:, :]),
                jnp.tile(b, 8).reshape(1, 256))

    _DIAG_JNP_PROP = False

    def prop(g):
        if _DIAG_JNP_PROP:
            gn = g.reshape(2 * ACC_ROWS, F)
            pl_ = jnp.zeros((ACC_ROWS, F), jnp.float32).at[dloc].add(gn[slo])
            ph_ = jnp.zeros((ACC_ROWS, F), jnp.float32).at[dloc].add(gn[shi])
            return pl_.reshape(RPK, 128), ph_.reshape(RPK, 128)
        plo, phi = _prop_call(g.reshape(2 * ACC_ROWS, F), slo, shi, dloc,
                              zeros)
        return plo.reshape(RPK, 128), phi.reshape(RPK, 128)

    plo, phi = prop(g)
    W0, W1a, W1b, b = wpk(c11_W0, c11_W1, c11_b)
    h2, g = _conv(h1, plo, phi, dinv128, dinv256, W0, W1a, W1b, b)

    plo, phi = prop(g)
    W0, W1a, W1b, b = wpk(c12_W0, c12_W1, c12_b)
    h3, g = _conv_res(h2, plo, phi, dinv128, dinv256, W0, W1a, W1b, b, h1)

    plo, phi = prop(g)
    W0, W1a, W1b, b = wpk(c21_W0, c21_W1, c21_b)
    h4, g = _conv(h3, plo, phi, dinv128, dinv256, W0, W1a, W1b, b)

    plo, phi = prop(g)
    W0, W1a, W1b, b = wpk(c22_W0, c22_W1, c22_b)
    lw = _kron8(lin1_W)
    lb = jnp.broadcast_to(lin1_b.reshape(1, 1), (1, 8))
    out_pk, = _conv_final(h4, plo, phi, dinv128, W0, W1a, W1b, b, h3, lw, lb)
    return out_pk.reshape(ACC_ROWS, 1)[:N]


def kernel(x, edge_index, lin0_W, lin0_b, c11_W0, c11_W1, c11_b,
           c12_W0, c12_W1, c12_b, c21_W0, c21_W1, c21_b,
           c22_W0, c22_W1, c22_b, lin1_W, lin1_b):
    return _run(x, edge_index, lin0_W, lin0_b, c11_W0, c11_W1, c11_b,
                c12_W0, c12_W1, c12_b, c21_W0, c21_W1, c21_b,
                c22_W0, c22_W1, c22_b, lin1_W, lin1_b)


# speedup vs baseline: 1.5427x; 1.1701x over previous
import jax
import jax.numpy as jnp
from jax import lax
from jax.experimental import pallas as pl
from jax.experimental.pallas import tpu as pltpu
from jax.experimental.pallas import tpu_sc as plsc

N = 100000
E = 1600000
L = 16
NC = 2
NS = 16
E_PAD = 1638400
ACC_ROWS = 100096
ZROWS = ACC_ROWS // NS
F = 16
CE = 640
RPK = ACC_ROWS // 8

_sc_mesh = plsc.VectorSubcoreMesh(core_axis_name="c", subcore_axis_name="s")


def _deg_body(srcp, dstp, ones, zeros, degp0, degp1, dloc_o,
              src_v, dst_v, sloc_v, dloc_v, ones_v, acc):
    c = lax.axis_index("c")
    s = lax.axis_index("s")
    wid = c * NS + s

    pltpu.sync_copy(zeros, acc.at[pl.ds(s * ZROWS, ZROWS)])
    pltpu.sync_copy(ones, ones_v)
    plsc.subcore_barrier()

    trash = jnp.full((L,), N, jnp.int32) + s

    edges_per_tile = E_PAD // (NC * NS)
    e_base = wid * edges_per_tile

    def chunk(it, carry):
        e0 = e_base + it * CE
        pltpu.sync_copy(srcp.at[pl.ds(e0, CE)], src_v)
        pltpu.sync_copy(dstp.at[pl.ds(e0, CE)], dst_v)
        for i in range(CE // L):
            sl = pl.ds(i * L, L)
            s16 = src_v[sl]
            d16 = dst_v[sl]
            eq = s16 == d16
            sloc_v[sl] = jnp.where(eq, trash, s16)
            dloc_v[sl] = jnp.where(eq, trash, d16)
        pltpu.sync_copy(dloc_v, dloc_o.at[pl.ds(e0, CE)])
        pltpu.sync_copy(ones_v, acc.at[sloc_v], add=True)
        return carry

    lax.fori_loop(0, edges_per_tile // CE, chunk, 0)

    plsc.subcore_barrier()

    @pl.when(c == 0)
    def _():
        pltpu.sync_copy(acc.at[pl.ds(s * ZROWS, ZROWS)],
                        degp0.at[pl.ds(s * ZROWS, ZROWS)])

    @pl.when(c == 1)
    def _():
        pltpu.sync_copy(acc.at[pl.ds(s * ZROWS, ZROWS)],
                        degp1.at[pl.ds(s * ZROWS, ZROWS)])


_deg_call = pl.kernel(
    _deg_body,
    out_type=(
        jax.ShapeDtypeStruct((ACC_ROWS, F), jnp.float32),
        jax.ShapeDtypeStruct((ACC_ROWS, F), jnp.float32),
        jax.ShapeDtypeStruct((E_PAD,), jnp.int32),
    ),
    mesh=_sc_mesh,
    compiler_params=pltpu.CompilerParams(use_tc_tiling_on_sc=False),
    scratch_types=[
        pltpu.VMEM((CE,), jnp.int32),
        pltpu.VMEM((CE,), jnp.int32),
        pltpu.VMEM((CE,), jnp.int32),
        pltpu.VMEM((CE,), jnp.int32),
        pltpu.VMEM((CE, F), jnp.float32),
        pltpu.VMEM_SHARED((ACC_ROWS, F), jnp.float32),
    ],
)


def _prop_body(glo_n, ghi_n, srcp, dlocp, zeros, plo, phi,
               src_v, dloc_v, rows_v, acc, sem_i, sem_g, sem_s):
    c = lax.axis_index("c")
    s = lax.axis_index("s")

    pltpu.sync_copy(zeros, acc.at[pl.ds(s * ZROWS, ZROWS)])
    plsc.subcore_barrier()

    edges_per_tile = E_PAD // NS
    e_base = s * edges_per_tile
    n_chunks = edges_per_tile // CE
    nb = n_chunks // 2

    def make_loop(g2n):
        idxp = srcp
        def e0_of(ck):
            return e_base + ck * CE

        def fire_idx(ck, p):
            return (pltpu.async_copy(idxp.at[pl.ds(e0_of(ck), CE)],
                                     src_v.at[p], sem_i),
                    pltpu.async_copy(dlocp.at[pl.ds(e0_of(ck), CE)],
                                    dloc_v.at[p], sem_i))

        def wait_idx_recon(ck, p):
            pltpu.make_async_copy(idxp.at[pl.ds(e0_of(ck), CE)],
                                  src_v.at[p], sem_i).wait()
            pltpu.make_async_copy(dlocp.at[pl.ds(e0_of(ck), CE)],
                                  dloc_v.at[p], sem_i).wait()

        def fire_gather(p):
            return pltpu.async_copy(g2n.at[src_v.at[p]], rows_v.at[p], sem_g)

        def fire_scatter(p):
            return pltpu.async_copy(rows_v.at[p], acc.at[dloc_v.at[p]],
                                    sem_s, add=True)

        def wait_scatter_recon(p):
            pltpu.make_async_copy(rows_v.at[p], acc.at[dloc_v.at[p]],
                                  sem_s).wait()

        fire_idx(0, 0)

        def body(q, carry):
            c0 = 2 * q
            wait_idx_recon(c0, 0)
            di1 = fire_idx(c0 + 1, 1)
            gd0 = fire_gather(0)

            @pl.when(q > 0)
            def _():
                wait_scatter_recon(1)
            gd0.wait()
            sd0 = fire_scatter(0)
            for d in di1:
                d.wait()

            @pl.when(q < nb - 1)
            def _():
                fire_idx(c0 + 2, 0)
            gd1 = fire_gather(1)
            sd0.wait()
            gd1.wait()
            fire_scatter(1)
            return carry

        lax.fori_loop(0, nb, body, 0)
        wait_scatter_recon(1)

    @pl.when(c == 0)
    def _():
        make_loop(glo_n)

    @pl.when(c == 1)
    def _():
        make_loop(ghi_n)

    plsc.subcore_barrier()

    @pl.when(c == 0)
    def _():
        pltpu.sync_copy(acc.at[pl.ds(s * ZROWS, ZROWS)],
                        plo.at[pl.ds(s * ZROWS, ZROWS)])

    @pl.when(c == 1)
    def _():
        pltpu.sync_copy(acc.at[pl.ds(s * ZROWS, ZROWS)],
                        phi.at[pl.ds(s * ZROWS, ZROWS)])


_prop_call = pl.kernel(
    _prop_body,
    out_type=(
        jax.ShapeDtypeStruct((ACC_ROWS, F), jnp.float32),
        jax.ShapeDtypeStruct((ACC_ROWS, F), jnp.float32),
    ),
    mesh=_sc_mesh,
    compiler_params=pltpu.CompilerParams(use_tc_tiling_on_sc=False),
    scratch_types=[
        pltpu.VMEM((2, CE), jnp.int32),
        pltpu.VMEM((2, CE), jnp.int32),
        pltpu.VMEM((2, CE, F), jnp.float32),
        pltpu.VMEM_SHARED((ACC_ROWS, F), jnp.float32),
        pltpu.SemaphoreType.DMA,
        pltpu.SemaphoreType.DMA,
        pltpu.SemaphoreType.DMA,
    ],
)


_BB = 736


def _rowspec(width):
    return pl.BlockSpec((_BB, width), lambda i: (i, 0))


def _whole(shape):
    return pl.BlockSpec(shape, lambda i: (0, 0))


def _dinv_body(degp0, degp1, dinv128_out):
    deg = degp0[...] + degp1[...]
    dinv128_out[...] = jnp.where(deg > 0,
                                 lax.rsqrt(jnp.where(deg > 0, deg, 1.0)), 0.0)


def _tc_dinv(degp0_pk, degp1_pk):
    return pl.pallas_call(
        _dinv_body,
        grid=(RPK // _BB,),
        in_specs=[pl.BlockSpec((_BB, 128), lambda i: (i, 0))] * 2,
        out_specs=pl.BlockSpec((_BB, 128), lambda i: (i, 0)),
        out_shape=jax.ShapeDtypeStruct((RPK, 128), jnp.float32),
    )(degp0_pk, degp1_pk)


def _split_g(g):
    g3 = g.reshape(g.shape[0], 8, 32)
    return (g3[:, :, :16].reshape(g.shape[0], 128),
            g3[:, :, 16:].reshape(g.shape[0], 128))


def _tc0_body(xpk, dinv256, Wx, b, h_out, glo_out, ghi_out):
    h = jnp.dot(xpk[...], Wx[...], preferred_element_type=jnp.float32) + b[...]
    h = jnp.maximum(h, 0.0)
    h_out[...] = h
    glo, ghi = _split_g(dinv256[...] * h)
    glo_out[...] = glo
    ghi_out[...] = ghi


def _tc0(xpk, dinv256, Wx, b):
    return pl.pallas_call(
        _tc0_body,
        grid=(RPK // _BB,),
        in_specs=[_rowspec(64), _rowspec(256), _whole((64, 256)),
                  _whole((1, 256))],
        out_specs=[_rowspec(256), _rowspec(128), _rowspec(128)],
        out_shape=[
            jax.ShapeDtypeStruct((RPK, 256), jnp.float32),
            jax.ShapeDtypeStruct((RPK, 128), jnp.float32),
            jax.ShapeDtypeStruct((RPK, 128), jnp.float32),
        ],
    )(xpk, dinv256, Wx, b)


def _conv_math(h, plo, phi, dinv128, W0, W1a, W1b, b, res):
    tlo = (-dinv128[...]) * plo[...]
    thi = (-dinv128[...]) * phi[...]
    acc = jnp.dot(h[...], W0[...], preferred_element_type=jnp.float32)
    acc = acc + jnp.dot(tlo, W1a[...], preferred_element_type=jnp.float32)
    acc = acc + jnp.dot(thi, W1b[...], preferred_element_type=jnp.float32)
    acc = acc + b[...]
    if res is not None:
        acc = acc + res[...]
    return jnp.maximum(acc, 0.0)


def _conv_body(h, plo, phi, dinv128, dinv256, W0, W1a, W1b, b,
               h_out, glo_out, ghi_out):
    hn = _conv_math(h, plo, phi, dinv128, W0, W1a, W1b, b, None)
    h_out[...] = hn
    glo, ghi = _split_g(dinv256[...] * hn)
    glo_out[...] = glo
    ghi_out[...] = ghi


def _conv_res_body(h, plo, phi, dinv128, dinv256, W0, W1a, W1b, b, res,
                   h_out, glo_out, ghi_out):
    hn = _conv_math(h, plo, phi, dinv128, W0, W1a, W1b, b, res)
    h_out[...] = hn
    glo, ghi = _split_g(dinv256[...] * hn)
    glo_out[...] = glo
    ghi_out[...] = ghi


def _conv_final_body(h, plo, phi, dinv128, W0, W1a, W1b, b, res, lw, lb,
                     out_pk):
    hn = _conv_math(h, plo, phi, dinv128, W0, W1a, W1b, b, res)
    out_pk[...] = jnp.dot(hn, lw[...],
                          preferred_element_type=jnp.float32) + lb[...]


_conv_common_in = [_rowspec(256), _rowspec(128), _rowspec(128),
                   _rowspec(128), _rowspec(256), _whole((256, 256)),
                   _whole((128, 256)), _whole((128, 256)), _whole((1, 256))]


def _conv(h, plo, phi, dinv128, dinv256, W0, W1a, W1b, b):
    return pl.pallas_call(
        _conv_body,
        grid=(RPK // _BB,),
        in_specs=_conv_common_in,
        out_specs=[_rowspec(256), _rowspec(128), _rowspec(128)],
        out_shape=[
            jax.ShapeDtypeStruct((RPK, 256), jnp.float32),
            jax.ShapeDtypeStruct((RPK, 128), jnp.float32),
            jax.ShapeDtypeStruct((RPK, 128), jnp.float32),
        ],
    )(h, plo, phi, dinv128, dinv256, W0, W1a, W1b, b)


def _conv_res(h, plo, phi, dinv128, dinv256, W0, W1a, W1b, b, res):
    return pl.pallas_call(
        _conv_res_body,
        grid=(RPK // _BB,),
        in_specs=_conv_common_in + [_rowspec(256)],
        out_specs=[_rowspec(256), _rowspec(128), _rowspec(128)],
        out_shape=[
            jax.ShapeDtypeStruct((RPK, 256), jnp.float32),
            jax.ShapeDtypeStruct((RPK, 128), jnp.float32),
            jax.ShapeDtypeStruct((RPK, 128), jnp.float32),
        ],
    )(h, plo, phi, dinv128, dinv256, W0, W1a, W1b, b, res)


def _conv_final(h, plo, phi, dinv128, W0, W1a, W1b, b, res, lw, lb):
    return pl.pallas_call(
        _conv_final_body,
        grid=(RPK // _BB,),
        in_specs=[_rowspec(256), _rowspec(128), _rowspec(128),
                  _rowspec(128), _whole((256, 256)), _whole((128, 256)),
                  _whole((128, 256)), _whole((1, 256)), _rowspec(256),
                  _whole((256, 8)), _whole((1, 8))],
        out_specs=[_rowspec(8)],
        out_shape=[jax.ShapeDtypeStruct((RPK, 8), jnp.float32)],
    )(h, plo, phi, dinv128, W0, W1a, W1b, b, res, lw, lb)


def _kron8(W):
    k, m = W.shape
    eye = jnp.eye(8, dtype=jnp.float32)
    return (eye[:, None, :, None] * W[None, :, None, :]).reshape(8 * k, 8 * m)


@jax.jit
def _run(x, edge_index, lin0_W, lin0_b, c11_W0, c11_W1, c11_b,
         c12_W0, c12_W1, c12_b, c21_W0, c21_W1, c21_b,
         c22_W0, c22_W1, c22_b, lin1_W, lin1_b):
    pad = E_PAD - E
    src1 = jnp.pad(edge_index[0], (0, pad))
    dst1 = jnp.pad(edge_index[1], (0, pad))
    ones = jnp.ones((CE, F), jnp.float32)
    zeros = jnp.zeros((ZROWS, F), jnp.float32)

    degp0, degp1, dloc = _deg_call(src1, dst1, ones, zeros)

    dinv128 = _tc_dinv(degp0.reshape(RPK, 128), degp1.reshape(RPK, 128))
    dinv_n = dinv128.reshape(ACC_ROWS, F)[:, :1]
    dinv256 = jnp.broadcast_to(dinv_n, (ACC_ROWS, 32)).reshape(RPK, 256)

    xpk = jnp.pad(x, ((0, ACC_ROWS - N), (0, 5))).reshape(RPK, 64)
    Wx = _kron8(jnp.pad(lin0_W, ((0, 5), (0, 0))))
    bpk = jnp.tile(lin0_b, 8).reshape(1, 256)
    h1, glo, ghi = _tc0(xpk, dinv256, Wx, bpk)

    def wpk(W0, W1, b):
        return (_kron8(W0), _kron8(W1[:F, :]), _kron8(W1[F:, :]),
                jnp.tile(b, 8).reshape(1, 256))

    def prop(glo, ghi):
        plo, phi = _prop_call(glo.reshape(ACC_ROWS, F),
                              ghi.reshape(ACC_ROWS, F), src1, dloc, zeros)
        return plo.reshape(RPK, 128), phi.reshape(RPK, 128)

    plo, phi = prop(glo, ghi)
    W0, W1a, W1b, b = wpk(c11_W0, c11_W1, c11_b)
    h2, glo, ghi = _conv(h1, plo, phi, dinv128, dinv256, W0, W1a, W1b, b)

    plo, phi = prop(glo, ghi)
    W0, W1a, W1b, b = wpk(c12_W0, c12_W1, c12_b)
    h3, glo, ghi = _conv_res(h2, plo, phi, dinv128, dinv256, W0, W1a, W1b, b,
                             h1)

    plo, phi = prop(glo, ghi)
    W0, W1a, W1b, b = wpk(c21_W0, c21_W1, c21_b)
    h4, glo, ghi = _conv(h3, plo, phi, dinv128, dinv256, W0, W1a, W1b, b)

    plo, phi = prop(glo, ghi)
    W0, W1a, W1b, b = wpk(c22_W0, c22_W1, c22_b)
    lw = _kron8(lin1_W)
    lb = jnp.broadcast_to(lin1_b.reshape(1, 1), (1, 8))
    out_pk, = _conv_final(h4, plo, phi, dinv128, W0, W1a, W1b, b, h3, lw, lb)
    return out_pk.reshape(ACC_ROWS, 1)[:N]


def kernel(x, edge_index, lin0_W, lin0_b, c11_W0, c11_W1, c11_b,
           c12_W0, c12_W1, c12_b, c21_W0, c21_W1, c21_b,
           c22_W0, c22_W1, c22_b, lin1_W, lin1_b):
    return _run(x, edge_index, lin0_W, lin0_b, c11_W0, c11_W1, c11_b,
                c12_W0, c12_W1, c12_b, c21_W0, c21_W1, c21_b,
                c22_W0, c22_W1, c22_b, lin1_W, lin1_b)
